# Initial kernel scaffold; baseline (speedup 1.0000x reference)
import jax
import jax.numpy as jnp
from jax.experimental import pallas as pl


def kernel(source_feat, edge_index, range_list, x_norm, embed, weight):
    raise NotImplementedError("write your pallas kernel here")


# speedup vs baseline: 10.9707x; 10.9707x over previous
import functools

import jax
import jax.numpy as jnp
from jax import lax
from jax.experimental import pallas as pl
from jax.experimental.pallas import tpu as pltpu
from jax.experimental.pallas import tpu_sc as plsc

N = 10000
E = 320000
D = 128
UNI_SRC = 8000
NT = 2000
DA = 144
L = 16
NC = 2
NS = 16
NW = NC * NS
EW = E // NW
C = 128
CHUNKS_CAP = (EW + C - 1) // C + 1
ACC_ROWS = 2048
DUMMY = ACC_ROWS - 1
RB = 2000


def _prologue_body(x_ref, xn_ref, emb_ref, out_ref):
    xm = jnp.dot(x_ref[...], emb_ref[...], preferred_element_type=jnp.float32)
    xm = xm / xn_ref[...]
    col = lax.broadcasted_iota(jnp.int32, (RB, DA - D), 1)
    ones_col = jnp.where(col == 0, 1.0, 0.0).astype(jnp.float32)
    out_ref[...] = jnp.concatenate([xm, ones_col], axis=1)


def _sc_body(xaug_hbm, src_hbm, dst_hbm, out_hbm,
             src_v, dst_v, cidx1, csrc1, cidx2, csrc2, rows_v, rem_v, acc):
    cid = lax.axis_index("c")
    sid = lax.axis_index("s")
    wid = sid * NC + cid

    def _zero_rows(i, _):
        for k in range(DA // L):
            rows_v[i, pl.ds(k * L, L)] = jnp.zeros((L,), jnp.float32)
        return 0

    @pl.when(sid == 0)
    def _():
        lax.fori_loop(0, C, _zero_rows, 0)

        def _zero_acc(t, _):
            pltpu.sync_copy(rows_v, acc.at[pl.ds(t * C, C)])
            return 0
        lax.fori_loop(0, ACC_ROWS // C, _zero_acc, 0)

    pltpu.sync_copy(src_hbm.at[pl.ds(wid * EW, EW)], src_v)
    pltpu.sync_copy(dst_hbm.at[pl.ds(wid * EW, EW)], dst_v)

    zv = jnp.zeros((L,), jnp.int32)
    rem_v[...] = zv

    def _compact(i, carry):
        base8, rem = carry
        base8 = pl.multiple_of(base8, 8)
        remv = rem_v[...]
        d = dst_v[pl.ds(i * L, L)]
        s = src_v[pl.ds(i * L, L)]
        m = d >= UNI_SRC
        mi = m.astype(jnp.int32)
        pos = plsc.cumsum(mi) - 1 + remv
        plsc.store_scatter(cidx1.at[pl.ds(base8, 32)], [pos], d - UNI_SRC,
                           mask=m)
        plsc.store_scatter(csrc1.at[pl.ds(base8, 32)], [pos], s, mask=m)
        tot = rem + jnp.sum(mi)
        totv = remv + plsc.all_reduce_population_count(m)
        rem_v[...] = totv & 7
        return (base8 + (tot & ~7), tot & 7)

    base8, rem = lax.fori_loop(
        0, EW // L, _compact, (jnp.int32(0), jnp.int32(0)))
    base8 = pl.multiple_of(base8, 8)
    nkeep = base8 + rem

    lane = lax.broadcasted_iota(jnp.int32, (L,), 0)
    remv = rem_v[...]
    for k in range(C // L + 1):
        ppos = lane + remv + (k * L)
        plsc.store_scatter(cidx1.at[pl.ds(base8, 160)], [ppos],
                           jnp.full((L,), DUMMY, jnp.int32))
        plsc.store_scatter(csrc1.at[pl.ds(base8, 160)], [ppos], zv)

    nchunks = (nkeep + (C - 1)) // C

    def _restage(j, _):
        for k in range(C // L):
            cidx2[j, pl.ds(k * L, L)] = cidx1[pl.ds(j * C + k * L, L)]
            csrc2[j, pl.ds(k * L, L)] = csrc1[pl.ds(j * C + k * L, L)]
        return 0

    lax.fori_loop(0, nchunks, _restage, 0)

    plsc.subcore_barrier()

    def _chunk(j, _):
        pltpu.sync_copy(xaug_hbm.at[csrc2.at[j]], rows_v)
        pltpu.sync_copy(rows_v, acc.at[cidx2.at[j]], add=True)
        return 0

    lax.fori_loop(0, nchunks, _chunk, 0)

    plsc.subcore_barrier()

    @pl.when(sid == 0)
    def _():
        pltpu.sync_copy(acc, out_hbm.at[cid])


def _epilogue_body(p_ref, w_ref, out_ref):
    p0 = p_ref[0]
    p1 = p_ref[1]
    sums = p0[:NT, :D] + p1[:NT, :D]
    cnt = jnp.sum(p0[:NT, D:DA] + p1[:NT, D:DA], axis=1, keepdims=True)
    aggr = sums / jnp.maximum(cnt, 1.0)
    out_ref[...] = jnp.dot(aggr, w_ref[...], preferred_element_type=jnp.float32)


def kernel(source_feat, edge_index, range_list, x_norm, embed, weight):
    del range_list

    x_aug = pl.pallas_call(
        _prologue_body,
        grid=(N // RB,),
        in_specs=[
            pl.BlockSpec((RB, D), lambda i: (i, 0)),
            pl.BlockSpec((RB, 1), lambda i: (i, 0)),
            pl.BlockSpec((D, D), lambda i: (0, 0)),
        ],
        out_specs=pl.BlockSpec((RB, DA), lambda i: (i, 0)),
        out_shape=jax.ShapeDtypeStruct((N, DA), jnp.float32),
    )(source_feat, x_norm.reshape(N, 1), embed)

    mesh = plsc.VectorSubcoreMesh(core_axis_name="c", subcore_axis_name="s")

    partials = pl.kernel(
        _sc_body,
        out_type=jax.ShapeDtypeStruct((NC, ACC_ROWS, DA), jnp.float32),
        mesh=mesh,
        scratch_types=[
            pltpu.VMEM((EW,), jnp.int32),
            pltpu.VMEM((EW,), jnp.int32),
            pltpu.VMEM((CHUNKS_CAP * C + C,), jnp.int32),
            pltpu.VMEM((CHUNKS_CAP * C + C,), jnp.int32),
            pltpu.VMEM((CHUNKS_CAP, C), jnp.int32),
            pltpu.VMEM((CHUNKS_CAP, C), jnp.int32),
            pltpu.VMEM((C, DA), jnp.float32),
            pltpu.VMEM((L,), jnp.int32),
            pltpu.VMEM_SHARED((ACC_ROWS, DA), jnp.float32),
        ],
        compiler_params=pltpu.CompilerParams(
            needs_layout_passes=False, use_tc_tiling_on_sc=False),
    )(x_aug, edge_index[0], edge_index[1])

    out = pl.pallas_call(
        _epilogue_body,
        out_shape=jax.ShapeDtypeStruct((NT, D), jnp.float32),
    )(partials, weight)
    return out


# speedup vs baseline: 12.0627x; 1.0995x over previous
import functools

import jax
import jax.numpy as jnp
from jax import lax
from jax.experimental import pallas as pl
from jax.experimental.pallas import tpu as pltpu
from jax.experimental.pallas import tpu_sc as plsc

N = 10000
E = 320000
D = 128
UNI_SRC = 8000
NT = 2000
DA = 144
L = 16
NC = 2
NS = 16
NW = NC * NS
EW = E // NW
C = 128
CHUNKS_CAP = (EW + C - 1) // C + 1
ACC_ROWS = 2048
DUMMY = ACC_ROWS - 1
RB = 2000


def _prologue_body(x_ref, xn_ref, emb_ref, out_ref):
    xm = jnp.dot(x_ref[...], emb_ref[...], preferred_element_type=jnp.float32)
    xm = xm / xn_ref[...]
    col = lax.broadcasted_iota(jnp.int32, (RB, DA - D), 1)
    ones_col = jnp.where(col == 0, 1.0, 0.0).astype(jnp.float32)
    out_ref[...] = jnp.concatenate([xm, ones_col], axis=1)


def _sc_body(xaug_hbm, src_hbm, dst_hbm, out_hbm,
             src_v, dst_v, cidx1, csrc1, cidx2, csrc2, rows_a, rows_b,
             pos_v, gsem, acc):
    cid = lax.axis_index("c")
    sid = lax.axis_index("s")
    wid = sid * NC + cid

    def _zero_rows(i, _):
        for k in range(DA // L):
            rows_a[i, pl.ds(k * L, L)] = jnp.zeros((L,), jnp.float32)
        return 0

    lax.fori_loop(0, C, _zero_rows, 0)
    pltpu.sync_copy(rows_a, acc.at[pl.ds(sid * C, C)])

    pltpu.sync_copy(src_hbm.at[pl.ds(wid * EW, EW)], src_v)
    pltpu.sync_copy(dst_hbm.at[pl.ds(wid * EW, EW)], dst_v)

    zv = jnp.zeros((L,), jnp.int32)
    pos_v[...] = zv

    def _compact(i, _):
        posv = pos_v[...]
        d = dst_v[pl.ds(i * L, L)]
        s = src_v[pl.ds(i * L, L)]
        m = d >= UNI_SRC
        pos = plsc.cumsum(m.astype(jnp.int32)) - 1 + posv
        plsc.store_scatter(cidx1, [pos], d - UNI_SRC, mask=m)
        plsc.store_scatter(csrc1, [pos], s, mask=m)
        pos_v[...] = posv + plsc.all_reduce_population_count(m)
        return 0

    lax.fori_loop(0, EW // L, _compact, 0)
    posv = pos_v[...]
    nkeep = jnp.max(posv)

    lane = lax.broadcasted_iota(jnp.int32, (L,), 0)
    for k in range(C // L):
        ppos = lane + posv + (k * L)
        plsc.store_scatter(cidx1, [ppos], jnp.full((L,), DUMMY, jnp.int32))
        plsc.store_scatter(csrc1, [ppos], zv)

    nchunks = (nkeep + (C - 1)) // C

    def _restage(j, _):
        for k in range(C // L):
            cidx2[j, pl.ds(k * L, L)] = cidx1[pl.ds(j * C + k * L, L)]
            csrc2[j, pl.ds(k * L, L)] = csrc1[pl.ds(j * C + k * L, L)]
        return 0

    lax.fori_loop(0, nchunks, _restage, 0)

    plsc.subcore_barrier()

    @pl.when(nchunks > 0)
    def _():
        pltpu.async_copy(xaug_hbm.at[csrc2.at[0]], rows_a, gsem)

    def _pair(g, _):
        for b in range(2):
            buf, obuf = (rows_a, rows_b) if b == 0 else (rows_b, rows_a)
            j = g * 2 + b

            @pl.when(j < nchunks)
            def _():
                pltpu.make_async_copy(xaug_hbm.at[csrc2.at[j]], buf,
                                      gsem).wait()

                @pl.when(j + 1 < nchunks)
                def _():
                    pltpu.async_copy(xaug_hbm.at[csrc2.at[j + 1]], obuf, gsem)

                pltpu.sync_copy(buf, acc.at[cidx2.at[j]], add=True)
        return 0

    lax.fori_loop(0, (nchunks + 1) // 2, _pair, 0)

    plsc.subcore_barrier()

    @pl.when(sid == 0)
    def _():
        pltpu.sync_copy(acc, out_hbm.at[cid])


def _epilogue_body(p_ref, w_ref, out_ref):
    p0 = p_ref[0]
    p1 = p_ref[1]
    sums = p0[:NT, :D] + p1[:NT, :D]
    cnt = jnp.sum(p0[:NT, D:DA] + p1[:NT, D:DA], axis=1, keepdims=True)
    aggr = sums / jnp.maximum(cnt, 1.0)
    out_ref[...] = jnp.dot(aggr, w_ref[...], preferred_element_type=jnp.float32)


def kernel(source_feat, edge_index, range_list, x_norm, embed, weight):
    del range_list

    x_aug = pl.pallas_call(
        _prologue_body,
        grid=(N // RB,),
        in_specs=[
            pl.BlockSpec((RB, D), lambda i: (i, 0)),
            pl.BlockSpec((RB, 1), lambda i: (i, 0)),
            pl.BlockSpec((D, D), lambda i: (0, 0)),
        ],
        out_specs=pl.BlockSpec((RB, DA), lambda i: (i, 0)),
        out_shape=jax.ShapeDtypeStruct((N, DA), jnp.float32),
    )(source_feat, x_norm.reshape(N, 1), embed)

    mesh = plsc.VectorSubcoreMesh(core_axis_name="c", subcore_axis_name="s")

    partials = pl.kernel(
        _sc_body,
        out_type=jax.ShapeDtypeStruct((NC, ACC_ROWS, DA), jnp.float32),
        mesh=mesh,
        scratch_types=[
            pltpu.VMEM((EW,), jnp.int32),
            pltpu.VMEM((EW,), jnp.int32),
            pltpu.VMEM((CHUNKS_CAP * C + C,), jnp.int32),
            pltpu.VMEM((CHUNKS_CAP * C + C,), jnp.int32),
            pltpu.VMEM((CHUNKS_CAP, C), jnp.int32),
            pltpu.VMEM((CHUNKS_CAP, C), jnp.int32),
            pltpu.VMEM((C, DA), jnp.float32),
            pltpu.VMEM((C, DA), jnp.float32),
            pltpu.VMEM((L,), jnp.int32),
            pltpu.SemaphoreType.DMA,
            pltpu.VMEM_SHARED((ACC_ROWS, DA), jnp.float32),
        ],
        compiler_params=pltpu.CompilerParams(
            needs_layout_passes=False, use_tc_tiling_on_sc=False),
    )(x_aug, edge_index[0], edge_index[1])

    out = pl.pallas_call(
        _epilogue_body,
        out_shape=jax.ShapeDtypeStruct((NT, D), jnp.float32),
    )(partials, weight)
    return out


# speedup vs baseline: 12.1730x; 1.0091x over previous
import functools

import jax
import jax.numpy as jnp
from jax import lax
from jax.experimental import pallas as pl
from jax.experimental.pallas import tpu as pltpu
from jax.experimental.pallas import tpu_sc as plsc

N = 10000
E = 320000
D = 128
UNI_SRC = 8000
NT = 2000
DA = 144
L = 16
NC = 2
NS = 16
NW = NC * NS
EW = E // NW
C = 128
CHUNKS_CAP = (EW + C - 1) // C + 1
ACC_ROWS = 2048
DUMMY = ACC_ROWS - 1
RB = 2000


def _prologue_body(x_ref, xn_ref, emb_ref, out_ref):
    xm = jnp.dot(x_ref[...], emb_ref[...], preferred_element_type=jnp.float32)
    xm = xm / xn_ref[...]
    col = lax.broadcasted_iota(jnp.int32, (RB, DA - D), 1)
    ones_col = jnp.where(col == 0, 1.0, 0.0).astype(jnp.float32)
    out_ref[...] = jnp.concatenate([xm, ones_col], axis=1)


def _sc_body(xaug_hbm, src_hbm, dst_hbm, out_hbm,
             src_v, dst_v, cidx1, csrc1, cidx2, rows_a, rows_b,
             pos_v, gsem, ssem_a, ssem_b, acc):
    cid = lax.axis_index("c")
    sid = lax.axis_index("s")
    wid = sid * NC + cid

    def _zero_rows(i, _):
        for k in range(DA // L):
            rows_a[i, pl.ds(k * L, L)] = jnp.zeros((L,), jnp.float32)
        return 0

    lax.fori_loop(0, C, _zero_rows, 0)
    pltpu.sync_copy(rows_a, acc.at[pl.ds(sid * C, C)])

    pltpu.sync_copy(src_hbm.at[pl.ds(wid * EW, EW)], src_v)
    pltpu.sync_copy(dst_hbm.at[pl.ds(wid * EW, EW)], dst_v)

    zv = jnp.zeros((L,), jnp.int32)
    pos_v[...] = zv

    def _compact(i, _):
        posv = pos_v[...]
        d = dst_v[pl.ds(i * L, L)]
        s = src_v[pl.ds(i * L, L)]
        m = d >= UNI_SRC
        pos = plsc.cumsum(m.astype(jnp.int32)) - 1 + posv
        plsc.store_scatter(cidx1, [pos], d - UNI_SRC, mask=m)
        plsc.store_scatter(csrc1, [pos], s, mask=m)
        pos_v[...] = posv + plsc.all_reduce_population_count(m)
        return 0

    lax.fori_loop(0, EW // L, _compact, 0)
    posv = pos_v[...]
    nkeep = jnp.max(posv)

    lane = lax.broadcasted_iota(jnp.int32, (L,), 0)
    for k in range(C // L):
        ppos = lane + posv + (k * L)
        plsc.store_scatter(cidx1, [ppos], jnp.full((L,), DUMMY, jnp.int32))
        plsc.store_scatter(csrc1, [ppos], zv)

    nchunks = (nkeep + (C - 1)) // C

    def _restage(j, _):
        for k in range(C // L):
            cidx2[j, pl.ds(k * L, L)] = cidx1[pl.ds(j * C + k * L, L)]
        return 0

    lax.fori_loop(0, nchunks, _restage, 0)

    plsc.subcore_barrier()

    def _gidx(j):
        return csrc1.at[pl.ds(pl.multiple_of(j * C, 8), C)]

    def _sidx(j):
        return cidx2.at[j]

    @pl.when(nchunks > 0)
    def _():
        pltpu.async_copy(xaug_hbm.at[_gidx(0)], rows_a, gsem)

    def _pair(g, _):
        for b in range(2):
            buf, obuf = (rows_a, rows_b) if b == 0 else (rows_b, rows_a)
            sbuf, sobuf = (ssem_a, ssem_b) if b == 0 else (ssem_b, ssem_a)
            j = g * 2 + b

            @pl.when(j < nchunks)
            def _():
                pltpu.make_async_copy(xaug_hbm.at[_gidx(j)], buf, gsem).wait()

                @pl.when(j + 1 < nchunks)
                def _():
                    @pl.when(j >= 1)
                    def _():
                        pltpu.make_async_copy(
                            obuf, acc.at[_sidx(j - 1)], sobuf).wait()
                    pltpu.async_copy(xaug_hbm.at[_gidx(j + 1)], obuf, gsem)

                pltpu.async_copy(buf, acc.at[_sidx(j)], sbuf, add=True)
        return 0

    lax.fori_loop(0, (nchunks + 1) // 2, _pair, 0)

    for b, sem in ((0, ssem_a), (1, ssem_b)):
        @pl.when((nchunks >= 1) & ((nchunks - 1) % 2 == b))
        def _():
            pltpu.make_async_copy(rows_a if b == 0 else rows_b,
                                  acc.at[_sidx(nchunks - 1)], sem).wait()

        @pl.when((nchunks >= 2) & ((nchunks - 2) % 2 == b))
        def _():
            pltpu.make_async_copy(rows_a if b == 0 else rows_b,
                                  acc.at[_sidx(nchunks - 2)], sem).wait()

    plsc.subcore_barrier()

    @pl.when(sid == 0)
    def _():
        pltpu.sync_copy(acc, out_hbm.at[cid])


def _epilogue_body(p_ref, w_ref, out_ref):
    p0 = p_ref[0]
    p1 = p_ref[1]
    sums = p0[:NT, :D] + p1[:NT, :D]
    cnt = jnp.sum(p0[:NT, D:DA] + p1[:NT, D:DA], axis=1, keepdims=True)
    aggr = sums / jnp.maximum(cnt, 1.0)
    out_ref[...] = jnp.dot(aggr, w_ref[...], preferred_element_type=jnp.float32)


def kernel(source_feat, edge_index, range_list, x_norm, embed, weight):
    del range_list

    x_aug = pl.pallas_call(
        _prologue_body,
        grid=(N // RB,),
        in_specs=[
            pl.BlockSpec((RB, D), lambda i: (i, 0)),
            pl.BlockSpec((RB, 1), lambda i: (i, 0)),
            pl.BlockSpec((D, D), lambda i: (0, 0)),
        ],
        out_specs=pl.BlockSpec((RB, DA), lambda i: (i, 0)),
        out_shape=jax.ShapeDtypeStruct((N, DA), jnp.float32),
    )(source_feat, x_norm.reshape(N, 1), embed)

    mesh = plsc.VectorSubcoreMesh(core_axis_name="c", subcore_axis_name="s")

    partials = pl.kernel(
        _sc_body,
        out_type=jax.ShapeDtypeStruct((NC, ACC_ROWS, DA), jnp.float32),
        mesh=mesh,
        scratch_types=[
            pltpu.VMEM((EW,), jnp.int32),
            pltpu.VMEM((EW,), jnp.int32),
            pltpu.VMEM((CHUNKS_CAP * C + C,), jnp.int32),
            pltpu.VMEM((CHUNKS_CAP * C + C,), jnp.int32),
            pltpu.VMEM((CHUNKS_CAP, C), jnp.int32),
            pltpu.VMEM((C, DA), jnp.float32),
            pltpu.VMEM((C, DA), jnp.float32),
            pltpu.VMEM((L,), jnp.int32),
            pltpu.SemaphoreType.DMA,
            pltpu.SemaphoreType.DMA,
            pltpu.SemaphoreType.DMA,
            pltpu.VMEM_SHARED((ACC_ROWS, DA), jnp.float32),
        ],
        compiler_params=pltpu.CompilerParams(
            needs_layout_passes=False, use_tc_tiling_on_sc=False),
    )(x_aug, edge_index[0], edge_index[1])

    out = pl.pallas_call(
        _epilogue_body,
        out_shape=jax.ShapeDtypeStruct((NT, D), jnp.float32),
    )(partials, weight)
    return out
